# baseline (speedup 1.0000x reference)
import functools

import jax
import jax.numpy as jnp
from jax.experimental import pallas as pl
from jax.experimental.pallas import tpu as pltpu

_N = 49
_E = 168
_F3 = 12 * _N
_MLP = 256

_O_W1L = 0
_O_W1R = 6
_O_B1 = 12
_O_W2L = 18
_O_W2R = 54
_O_B2 = 90
_O_W3L = 96
_O_W3R = 168
_O_B3 = 240
_O_BF3 = 252
_O_BF1 = 256
_O_BF2 = 512
_O_WF3 = 768
_O_SRC = 1024
_O_DST = 1192
_PACK = 1408


def _dot_t(x, y):
    return jax.lax.dot_general(x, y, (((1,), (1,)), ((), ())),
                               preferred_element_type=jnp.float32)


def _dot_tl(x, y):
    return jax.lax.dot_general(x, y, (((0,), (0,)), ((), ())),
                               preferred_element_type=jnp.float32)


def _iota2(shape, dim):
    return jax.lax.broadcasted_iota(jnp.int32, shape, dim)


def _fiota2(shape, dim):
    return jax.lax.broadcasted_iota(jnp.int32, shape, dim).astype(jnp.float32)


def _fused_net(xa_ref, pk_ref, wf1_ref, wf2_ref, out_ref,
               m1_s, b1_s, m2_s, b2_s, g_s, wa_s, c_s):
    f32 = jnp.float32

    @pl.when(pl.program_id(0) == 0)
    def _prep():
        def seg(off, n):
            return pk_ref[0:1, off:off + n]

        dmat = (_fiota2((_N, _E), 0) == seg(_O_DST, _E)).astype(f32)
        smat = (_fiota2((_N, _E), 0) == seg(_O_SRC, _E)).astype(f32)
        adj = _dot_t(dmat, smat)
        deg = jnp.sum(adj, axis=1, keepdims=True)
        adj = adj / jnp.maximum(deg, 1.0)

        pt6 = (_iota2((_N, 6 * _N), 1) // 6 == _iota2((_N, 6 * _N), 0)).astype(f32)
        qt6 = (_iota2((6, 6 * _N), 1) % 6 == _iota2((6, 6 * _N), 0)).astype(f32)
        qt12 = (_iota2((12, _F3), 1) % 12 == _iota2((12, _F3), 0)).astype(f32)

        def kron_w(off, gdim, cols):
            t = 6 * gdim
            s = seg(off, t)
            t1 = (_iota2((6 * _N, t), 1) % 6
                  == _iota2((6 * _N, t), 0) % 6).astype(f32)
            t2 = (_iota2((t, cols), 0) // 6
                  == _iota2((t, cols), 1) % gdim).astype(f32)
            return jnp.dot(t1 * s, t2, preferred_element_type=f32)

        ka1 = _dot_tl(adj, pt6)
        w1l_row = jnp.dot(seg(_O_W1L, 6), qt6, preferred_element_type=f32)
        w1r_row = jnp.dot(seg(_O_W1R, 6), qt6, preferred_element_type=f32)
        m1_s[...] = ka1 * w1l_row + pt6 * w1r_row
        b1_s[...] = jnp.dot(seg(_O_B1, 6), qt6, preferred_element_type=f32)

        ka2 = _dot_tl(pt6, ka1)
        bm6 = (_iota2((6 * _N, 6 * _N), 0) // 6
               == _iota2((6 * _N, 6 * _N), 1) // 6).astype(f32)
        m2_s[...] = (ka2 * kron_w(_O_W2L, 6, 6 * _N)
                     + bm6 * kron_w(_O_W2R, 6, 6 * _N))
        b2_s[...] = jnp.dot(seg(_O_B2, 6), qt6, preferred_element_type=f32)

        pt12 = (_iota2((_N, _F3), 1) // 12 == _iota2((_N, _F3), 0)).astype(f32)
        ka3 = _dot_tl(pt6, _dot_tl(adj, pt12))
        bm612 = (_iota2((6 * _N, _F3), 0) // 6
                 == _iota2((6 * _N, _F3), 1) // 12).astype(f32)
        m3 = ka3 * kron_w(_O_W3L, 12, _F3) + bm612 * kron_w(_O_W3R, 12, _F3)
        wf1m = wf1_ref[:, :_F3]
        g_s[...] = _dot_t(m3, wf1m)
        b3_row = jnp.dot(seg(_O_B3, 12), qt12, preferred_element_type=f32)
        c_s[...] = _dot_t(b3_row, wf1m) + seg(_O_BF1, _MLP)
        wa_s[...] = jax.lax.dot_general(
            jnp.ones((1, 1), f32), wf1_ref[:, _F3:],
            (((0,), (1,)), ((), ())), preferred_element_type=f32)

    h0 = xa_ref[:, :_N]
    av = xa_ref[:, _N:_N + 1]
    h1 = jnp.maximum(
        jnp.dot(h0, m1_s[...], preferred_element_type=jnp.float32)
        + b1_s[...], 0.0)
    h2 = jnp.maximum(
        jnp.dot(h1, m2_s[...], preferred_element_type=jnp.float32)
        + b2_s[...], 0.0)
    f1 = jnp.maximum(
        jnp.dot(h2, g_s[...], preferred_element_type=jnp.float32)
        + av * wa_s[...] + c_s[...], 0.0)
    f2 = jnp.maximum(
        _dot_t(f1, wf2_ref[...]) + pk_ref[0:1, _O_BF2:_O_BF2 + _MLP], 0.0)
    ovec = (jnp.sum(f2 * pk_ref[0:1, _O_WF3:_O_WF3 + _MLP], axis=1,
                    keepdims=True)
            + pk_ref[0:1, _O_BF3:_O_BF3 + 1])
    out_ref[...] = jnp.reshape(ovec, (ovec.shape[0] // 128, 128))


@functools.partial(jax.jit, static_argnames=())
def kernel(x, a, edge_index, W1l, W1r, b1, W2l, W2r, b2, W3l, W3r, b3,
           Wf1, bf1, Wf2, bf2, Wf3, bf3):
    B = x.shape[0]
    f32 = jnp.float32
    xa = jnp.concatenate([x.reshape(B, _N), a], axis=1)
    ei = edge_index.astype(f32)
    pack = jnp.concatenate([
        W1l.ravel(), W1r.ravel(), b1, W2l.ravel(), W2r.ravel(), b2,
        W3l.ravel(), W3r.ravel(), b3, bf3, jnp.zeros((3,), f32),
        bf1, bf2, Wf3.ravel(), ei[0], ei[1],
        jnp.zeros((_PACK - _O_DST - _E,), f32),
    ])[None, :]

    bb = 4096
    grid = (B // bb,)

    def full(arr):
        return pl.BlockSpec(arr.shape, lambda i: tuple(0 for _ in arr.shape))

    out2 = pl.pallas_call(
        _fused_net,
        grid=grid,
        in_specs=[
            pl.BlockSpec((bb, _N + 1), lambda i: (i, 0)),
            full(pack), full(Wf1), full(Wf2),
        ],
        out_specs=pl.BlockSpec((bb // 128, 128), lambda i: (i, 0)),
        out_shape=jax.ShapeDtypeStruct((B // 128, 128), f32),
        scratch_shapes=[
            pltpu.VMEM((_N, 6 * _N), f32),
            pltpu.VMEM((1, 6 * _N), f32),
            pltpu.VMEM((6 * _N, 6 * _N), f32),
            pltpu.VMEM((1, 6 * _N), f32),
            pltpu.VMEM((6 * _N, _MLP), f32),
            pltpu.VMEM((1, _MLP), f32),
            pltpu.VMEM((1, _MLP), f32),
        ],
    )(xa, pack, Wf1, Wf2)
    return out2.reshape(B, 1)
